# Initial kernel scaffold; baseline (speedup 1.0000x reference)
import jax
import jax.numpy as jnp
from jax.experimental import pallas as pl


def kernel(x, patch_w_t, patch_b, cls_full, pos_patch, norm_g, norm_b, blk00_ln1_g, blk00_ln1_b, blk00_qkv_w_t, blk00_qkv_b, blk00_proj_w_t, blk00_proj_b, blk00_ln2_g, blk00_ln2_b, blk00_fc1_w_t, blk00_fc1_b, blk00_fc2_w_t, blk00_fc2_b, blk01_ln1_g, blk01_ln1_b, blk01_qkv_w_t, blk01_qkv_b, blk01_proj_w_t, blk01_proj_b, blk01_ln2_g, blk01_ln2_b, blk01_fc1_w_t, blk01_fc1_b, blk01_fc2_w_t, blk01_fc2_b, blk02_ln1_g, blk02_ln1_b, blk02_qkv_w_t, blk02_qkv_b, blk02_proj_w_t, blk02_proj_b, blk02_ln2_g, blk02_ln2_b, blk02_fc1_w_t, blk02_fc1_b, blk02_fc2_w_t, blk02_fc2_b, blk03_ln1_g, blk03_ln1_b, blk03_qkv_w_t, blk03_qkv_b, blk03_proj_w_t, blk03_proj_b, blk03_ln2_g, blk03_ln2_b, blk03_fc1_w_t, blk03_fc1_b, blk03_fc2_w_t, blk03_fc2_b, blk04_ln1_g, blk04_ln1_b, blk04_qkv_w_t, blk04_qkv_b, blk04_proj_w_t, blk04_proj_b, blk04_ln2_g, blk04_ln2_b, blk04_fc1_w_t, blk04_fc1_b, blk04_fc2_w_t, blk04_fc2_b, blk05_ln1_g, blk05_ln1_b, blk05_qkv_w_t, blk05_qkv_b, blk05_proj_w_t, blk05_proj_b, blk05_ln2_g, blk05_ln2_b, blk05_fc1_w_t, blk05_fc1_b, blk05_fc2_w_t, blk05_fc2_b, blk06_ln1_g, blk06_ln1_b, blk06_qkv_w_t, blk06_qkv_b, blk06_proj_w_t, blk06_proj_b, blk06_ln2_g, blk06_ln2_b, blk06_fc1_w_t, blk06_fc1_b, blk06_fc2_w_t, blk06_fc2_b, blk07_ln1_g, blk07_ln1_b, blk07_qkv_w_t, blk07_qkv_b, blk07_proj_w_t, blk07_proj_b, blk07_ln2_g, blk07_ln2_b, blk07_fc1_w_t, blk07_fc1_b, blk07_fc2_w_t, blk07_fc2_b, blk08_ln1_g, blk08_ln1_b, blk08_qkv_w_t, blk08_qkv_b, blk08_proj_w_t, blk08_proj_b, blk08_ln2_g, blk08_ln2_b, blk08_fc1_w_t, blk08_fc1_b, blk08_fc2_w_t, blk08_fc2_b, blk09_ln1_g, blk09_ln1_b, blk09_qkv_w_t, blk09_qkv_b, blk09_proj_w_t, blk09_proj_b, blk09_ln2_g, blk09_ln2_b, blk09_fc1_w_t, blk09_fc1_b, blk09_fc2_w_t, blk09_fc2_b, blk10_ln1_g, blk10_ln1_b, blk10_qkv_w_t, blk10_qkv_b, blk10_proj_w_t, blk10_proj_b, blk10_ln2_g, blk10_ln2_b, blk10_fc1_w_t, blk10_fc1_b, blk10_fc2_w_t, blk10_fc2_b, blk11_ln1_g, blk11_ln1_b, blk11_qkv_w_t, blk11_qkv_b, blk11_proj_w_t, blk11_proj_b, blk11_ln2_g, blk11_ln2_b, blk11_fc1_w_t, blk11_fc1_b, blk11_fc2_w_t, blk11_fc2_b):
    raise NotImplementedError("write your pallas kernel here")


# speedup vs baseline: 1.0134x; 1.0134x over previous
import functools
import math

import jax
import jax.numpy as jnp
from jax import lax
from jax.experimental import pallas as pl
from jax.experimental.pallas import tpu as pltpu

_MIB = 1024 * 1024


def _vit_megakernel(xp_ref, pw_ref, pb_ref, pos_ref, cls_ref, ng_ref, nb_ref,
                    ln1g_ref, ln1b_ref, qkvw_ref, qkvb_ref,
                    projw_ref, projb_ref, ln2g_ref, ln2b_ref,
                    fc1w_ref, fc1b_ref, fc2w_ref, fc2b_ref,
                    o_ref, scr, attn_scr,
                    *, G, nh, hd, depth, n_mlp_chunks, eps):
    d = pl.program_id(1)
    D = nh * hd
    scale = hd ** -0.5

    @pl.when(d == 0)
    def _embed():
        def emb(i, _):
            y = jnp.dot(xp_ref[i], pw_ref[...],
                        preferred_element_type=jnp.float32)
            scr[i, 0:1, :] = cls_ref[...]
            scr[i, 1:, :] = y + pb_ref[...] + pos_ref[0]
            return 0
        lax.fori_loop(0, G, emb, 0, unroll=True)

    ln1g = ln1g_ref[0]
    ln1b = ln1b_ref[0]
    wqkv = qkvw_ref[0]
    bqkv = qkvb_ref[0]
    wproj = projw_ref[0]
    bproj = projb_ref[0]
    ln2g = ln2g_ref[0]
    ln2b = ln2b_ref[0]
    w1 = fc1w_ref[0]
    b1 = fc1b_ref[0]
    w2 = fc2w_ref[0]
    b2 = fc2b_ref[0]
    Hc = w1.shape[1] // n_mlp_chunks

    def body(i, _):
        x = scr[i]
        mu = jnp.mean(x, axis=-1, keepdims=True)
        var = jnp.mean((x - mu) ** 2, axis=-1, keepdims=True)
        xb = ((x - mu) * lax.rsqrt(var + eps) * ln1g + ln1b
              ).astype(jnp.bfloat16)

        qkv = jnp.dot(xb, wqkv, preferred_element_type=jnp.float32) + bqkv
        qs = (qkv[:, :D] * scale).astype(jnp.bfloat16)
        kb = qkv[:, D:2 * D].astype(jnp.bfloat16)
        vb = qkv[:, 2 * D:].astype(jnp.bfloat16)

        for h in range(nh):
            lo, hi = h * hd, (h + 1) * hd
            s = lax.dot_general(qs[:, lo:hi], kb[:, lo:hi],
                                (((1,), (1,)), ((), ())),
                                preferred_element_type=jnp.float32)
            s = s - jnp.max(s, axis=-1, keepdims=True)
            p = jnp.exp(s)
            l = jnp.sum(p, axis=-1, keepdims=True)
            pv = jnp.dot(p.astype(jnp.bfloat16), vb[:, lo:hi],
                         preferred_element_type=jnp.float32)
            attn_scr[:, lo:hi] = (pv * pl.reciprocal(l, approx=True)
                                  ).astype(jnp.bfloat16)

        r1 = (x + jnp.dot(attn_scr[...], wproj,
                          preferred_element_type=jnp.float32) + bproj)

        mu2 = jnp.mean(r1, axis=-1, keepdims=True)
        var2 = jnp.mean((r1 - mu2) ** 2, axis=-1, keepdims=True)
        yn = ((r1 - mu2) * lax.rsqrt(var2 + eps) * ln2g + ln2b
              ).astype(jnp.bfloat16)

        acc = r1 + b2
        for c in range(n_mlp_chunks):
            lo, hi = c * Hc, (c + 1) * Hc
            hid = (jnp.dot(yn, w1[:, lo:hi],
                           preferred_element_type=jnp.float32) + b1[:, lo:hi])
            hid = 0.5 * hid * (1.0 + lax.erf(hid * (1.0 / math.sqrt(2.0))))
            acc = acc + jnp.dot(hid.astype(jnp.bfloat16), w2[lo:hi, :],
                                preferred_element_type=jnp.float32)
        scr[i] = acc
        return 0

    lax.fori_loop(0, G, body, 0)

    @pl.when(d == depth - 1)
    def _final():
        def fin(i, _):
            xc = scr[i, 0:1, :]
            mu = jnp.mean(xc, axis=-1, keepdims=True)
            var = jnp.mean((xc - mu) ** 2, axis=-1, keepdims=True)
            o_ref[i] = ((xc - mu) * lax.rsqrt(var + eps) * ng_ref[...]
                        + nb_ref[...])
            return 0
        lax.fori_loop(0, G, fin, 0, unroll=True)


def _vit_forward(xp, patch_w_t, patch_b, pos_patch, cls_full, norm_g, norm_b,
                 stk, *, nh, depth, G, n_mlp_chunks=4, eps=1e-6):
    B, nP, K = xp.shape
    D = patch_w_t.shape[1]
    Hd = stk["fc1_w"].shape[2]
    hd = D // nh
    N = nP + 1
    nc = B // G

    kern = functools.partial(_vit_megakernel, G=G, nh=nh, hd=hd, depth=depth,
                             n_mlp_chunks=n_mlp_chunks, eps=eps)
    fixed = lambda c, d: (0, 0)
    fixed3 = lambda c, d: (0, 0, 0)
    perblk = lambda c, d: (d, 0, 0)

    out = pl.pallas_call(
        kern,
        out_shape=jax.ShapeDtypeStruct((B, 1, D), jnp.float32),
        grid=(nc, depth),
        in_specs=[
            pl.BlockSpec((G, nP, K), lambda c, d: (c, 0, 0)),
            pl.BlockSpec((K, D), fixed),
            pl.BlockSpec((1, D), fixed),
            pl.BlockSpec((1, nP, D), fixed3),
            pl.BlockSpec((1, D), fixed),
            pl.BlockSpec((1, D), fixed),
            pl.BlockSpec((1, D), fixed),
            pl.BlockSpec((1, 1, D), perblk),
            pl.BlockSpec((1, 1, D), perblk),
            pl.BlockSpec((1, D, 3 * D), perblk),
            pl.BlockSpec((1, 1, 3 * D), perblk),
            pl.BlockSpec((1, D, D), perblk),
            pl.BlockSpec((1, 1, D), perblk),
            pl.BlockSpec((1, 1, D), perblk),
            pl.BlockSpec((1, 1, D), perblk),
            pl.BlockSpec((1, D, Hd), perblk),
            pl.BlockSpec((1, 1, Hd), perblk),
            pl.BlockSpec((1, Hd, D), perblk),
            pl.BlockSpec((1, 1, D), perblk),
        ],
        out_specs=pl.BlockSpec((G, 1, D), lambda c, d: (c, 0, 0)),
        scratch_shapes=[
            pltpu.VMEM((G, N, D), jnp.float32),
            pltpu.VMEM((N, D), jnp.bfloat16),
        ],
        compiler_params=pltpu.CompilerParams(
            dimension_semantics=("parallel", "arbitrary"),
            vmem_limit_bytes=56 * _MIB,
        ),
    )(xp, patch_w_t, patch_b, pos_patch, cls_full, norm_g, norm_b,
      stk["ln1_g"], stk["ln1_b"], stk["qkv_w"], stk["qkv_b"],
      stk["proj_w"], stk["proj_b"], stk["ln2_g"], stk["ln2_b"],
      stk["fc1_w"], stk["fc1_b"], stk["fc2_w"], stk["fc2_b"])
    return out[:, 0, :]


def kernel(x, patch_w_t, patch_b, cls_full, pos_patch, norm_g, norm_b, blk00_ln1_g, blk00_ln1_b, blk00_qkv_w_t, blk00_qkv_b, blk00_proj_w_t, blk00_proj_b, blk00_ln2_g, blk00_ln2_b, blk00_fc1_w_t, blk00_fc1_b, blk00_fc2_w_t, blk00_fc2_b, blk01_ln1_g, blk01_ln1_b, blk01_qkv_w_t, blk01_qkv_b, blk01_proj_w_t, blk01_proj_b, blk01_ln2_g, blk01_ln2_b, blk01_fc1_w_t, blk01_fc1_b, blk01_fc2_w_t, blk01_fc2_b, blk02_ln1_g, blk02_ln1_b, blk02_qkv_w_t, blk02_qkv_b, blk02_proj_w_t, blk02_proj_b, blk02_ln2_g, blk02_ln2_b, blk02_fc1_w_t, blk02_fc1_b, blk02_fc2_w_t, blk02_fc2_b, blk03_ln1_g, blk03_ln1_b, blk03_qkv_w_t, blk03_qkv_b, blk03_proj_w_t, blk03_proj_b, blk03_ln2_g, blk03_ln2_b, blk03_fc1_w_t, blk03_fc1_b, blk03_fc2_w_t, blk03_fc2_b, blk04_ln1_g, blk04_ln1_b, blk04_qkv_w_t, blk04_qkv_b, blk04_proj_w_t, blk04_proj_b, blk04_ln2_g, blk04_ln2_b, blk04_fc1_w_t, blk04_fc1_b, blk04_fc2_w_t, blk04_fc2_b, blk05_ln1_g, blk05_ln1_b, blk05_qkv_w_t, blk05_qkv_b, blk05_proj_w_t, blk05_proj_b, blk05_ln2_g, blk05_ln2_b, blk05_fc1_w_t, blk05_fc1_b, blk05_fc2_w_t, blk05_fc2_b, blk06_ln1_g, blk06_ln1_b, blk06_qkv_w_t, blk06_qkv_b, blk06_proj_w_t, blk06_proj_b, blk06_ln2_g, blk06_ln2_b, blk06_fc1_w_t, blk06_fc1_b, blk06_fc2_w_t, blk06_fc2_b, blk07_ln1_g, blk07_ln1_b, blk07_qkv_w_t, blk07_qkv_b, blk07_proj_w_t, blk07_proj_b, blk07_ln2_g, blk07_ln2_b, blk07_fc1_w_t, blk07_fc1_b, blk07_fc2_w_t, blk07_fc2_b, blk08_ln1_g, blk08_ln1_b, blk08_qkv_w_t, blk08_qkv_b, blk08_proj_w_t, blk08_proj_b, blk08_ln2_g, blk08_ln2_b, blk08_fc1_w_t, blk08_fc1_b, blk08_fc2_w_t, blk08_fc2_b, blk09_ln1_g, blk09_ln1_b, blk09_qkv_w_t, blk09_qkv_b, blk09_proj_w_t, blk09_proj_b, blk09_ln2_g, blk09_ln2_b, blk09_fc1_w_t, blk09_fc1_b, blk09_fc2_w_t, blk09_fc2_b, blk10_ln1_g, blk10_ln1_b, blk10_qkv_w_t, blk10_qkv_b, blk10_proj_w_t, blk10_proj_b, blk10_ln2_g, blk10_ln2_b, blk10_fc1_w_t, blk10_fc1_b, blk10_fc2_w_t, blk10_fc2_b, blk11_ln1_g, blk11_ln1_b, blk11_qkv_w_t, blk11_qkv_b, blk11_proj_w_t, blk11_proj_b, blk11_ln2_g, blk11_ln2_b, blk11_fc1_w_t, blk11_fc1_b, blk11_fc2_w_t, blk11_fc2_b):
    cfg_img, cfg_patch, cfg_cin = 224, 16, 3
    depth, nh = 12, 12
    B = x.shape[0]
    p = cfg_patch

    lcl = locals()
    names = ["ln1_g", "ln1_b", "qkv_w_t", "qkv_b", "proj_w_t", "proj_b",
             "ln2_g", "ln2_b", "fc1_w_t", "fc1_b", "fc2_w_t", "fc2_b"]
    blocks = [{n: lcl["blk%02d_%s" % (li, n)] for n in names}
              for li in range(depth)]

    stk = {
        "ln1_g": jnp.stack([b["ln1_g"] for b in blocks]),
        "ln1_b": jnp.stack([b["ln1_b"] for b in blocks]),
        "qkv_w": jnp.stack([b["qkv_w_t"] for b in blocks]),
        "qkv_b": jnp.stack([b["qkv_b"] for b in blocks]),
        "proj_w": jnp.stack([b["proj_w_t"] for b in blocks]),
        "proj_b": jnp.stack([b["proj_b"] for b in blocks]),
        "ln2_g": jnp.stack([b["ln2_g"] for b in blocks]),
        "ln2_b": jnp.stack([b["ln2_b"] for b in blocks]),
        "fc1_w": jnp.stack([b["fc1_w_t"] for b in blocks]),
        "fc1_b": jnp.stack([b["fc1_b"] for b in blocks]),
        "fc2_w": jnp.stack([b["fc2_w_t"] for b in blocks]),
        "fc2_b": jnp.stack([b["fc2_b"] for b in blocks]),
    }

    H = W = cfg_img
    xp = x.reshape(B, cfg_cin, H // p, p, W // p, p)
    xp = xp.transpose(0, 2, 4, 1, 3, 5).reshape(
        B, (H // p) * (W // p), cfg_cin * p * p).astype(jnp.bfloat16)

    return _vit_forward(xp, patch_w_t, patch_b, pos_patch, cls_full,
                        norm_g, norm_b, stk, nh=nh, depth=depth, G=8)


# speedup vs baseline: 1.0773x; 1.0631x over previous
import functools
import math

import jax
import jax.numpy as jnp
from jax import lax
from jax.experimental import pallas as pl
from jax.experimental.pallas import tpu as pltpu

_MIB = 1024 * 1024


def _vit_megakernel(xp_ref, pw_ref, pb_ref, pos_ref, cls_ref, ng_ref, nb_ref,
                    ln1g_ref, ln1b_ref, qkvw_ref, qkvb_ref,
                    projw_ref, projb_ref, ln2g_ref, ln2b_ref,
                    fc1w_ref, fc1b_ref, fc2w_ref, fc2b_ref,
                    o_ref, scr,
                    *, G, nh, hd, depth, n_mlp_chunks, eps):
    d = pl.program_id(1)
    D = nh * hd

    @pl.when(d == 0)
    def _embed():
        def emb(i, _):
            y = jnp.dot(xp_ref[i], pw_ref[...],
                        preferred_element_type=jnp.float32)
            scr[i, 0:1, :] = cls_ref[...]
            scr[i, 1:, :] = y + pb_ref[...] + pos_ref[0]
            return 0
        lax.fori_loop(0, G, emb, 0, unroll=True)

    ln1g = ln1g_ref[0]
    ln1b = ln1b_ref[0]
    wqkv = qkvw_ref[0]
    bqkv = qkvb_ref[0]
    wproj = projw_ref[0]
    bproj = projb_ref[0]
    ln2g = ln2g_ref[0]
    ln2b = ln2b_ref[0]
    w1 = fc1w_ref[0]
    b1 = fc1b_ref[0]
    w2 = fc2w_ref[0]
    b2 = fc2b_ref[0]
    Hc = w1.shape[1] // n_mlp_chunks

    inv_D = 1.0 / D

    def _ln(x, g, b):
        mu = jnp.sum(x, axis=-1, keepdims=True) * inv_D
        ms = jnp.sum(x * x, axis=-1, keepdims=True) * inv_D
        var = ms - mu * mu
        return (x - mu) * lax.rsqrt(var + eps) * g + b

    def body(i, _):
        x = scr[i]
        xb = _ln(x, ln1g, ln1b).astype(jnp.bfloat16)

        qkv = jnp.dot(xb, wqkv, preferred_element_type=jnp.float32) + bqkv
        qs = qkv[:, :D].astype(jnp.bfloat16)
        kb = qkv[:, D:2 * D].astype(jnp.bfloat16)
        vb = qkv[:, 2 * D:].astype(jnp.bfloat16)

        heads = []
        for h in range(nh):
            lo, hi = h * hd, (h + 1) * hd
            s = lax.dot_general(qs[:, lo:hi], kb[:, lo:hi],
                                (((1,), (1,)), ((), ())),
                                preferred_element_type=jnp.float32)
            s = s - jnp.max(s, axis=-1, keepdims=True)
            p = jnp.exp(s)
            l = jnp.sum(p, axis=-1, keepdims=True)
            pv = jnp.dot(p.astype(jnp.bfloat16), vb[:, lo:hi],
                         preferred_element_type=jnp.float32)
            heads.append((pv * pl.reciprocal(l, approx=True)
                          ).astype(jnp.bfloat16))
        attn = jnp.concatenate(heads, axis=1)

        r1 = (x + jnp.dot(attn, wproj,
                          preferred_element_type=jnp.float32) + bproj)

        yn = _ln(r1, ln2g, ln2b).astype(jnp.bfloat16)

        acc = r1 + b2
        for c in range(n_mlp_chunks):
            lo, hi = c * Hc, (c + 1) * Hc
            hid = (jnp.dot(yn, w1[:, lo:hi],
                           preferred_element_type=jnp.float32) + b1[:, lo:hi])
            hid = 0.5 * hid * (1.0 + lax.erf(hid * (1.0 / math.sqrt(2.0))))
            acc = acc + jnp.dot(hid.astype(jnp.bfloat16), w2[lo:hi, :],
                                preferred_element_type=jnp.float32)
        scr[i] = acc
        return 0

    lax.fori_loop(0, G, body, 0, unroll=2)

    @pl.when(d == depth - 1)
    def _final():
        def fin(i, _):
            xc = scr[i, 0:1, :]
            mu = jnp.mean(xc, axis=-1, keepdims=True)
            var = jnp.mean((xc - mu) ** 2, axis=-1, keepdims=True)
            o_ref[i] = ((xc - mu) * lax.rsqrt(var + eps) * ng_ref[...]
                        + nb_ref[...])
            return 0
        lax.fori_loop(0, G, fin, 0, unroll=True)


def _vit_forward(xp, patch_w_t, patch_b, pos_patch, cls_full, norm_g, norm_b,
                 stk, *, nh, depth, G, n_mlp_chunks=4, eps=1e-6):
    B, nP, K = xp.shape
    D = patch_w_t.shape[1]
    Hd = stk["fc1_w"].shape[2]
    hd = D // nh
    N = nP + 1
    nc = B // G

    kern = functools.partial(_vit_megakernel, G=G, nh=nh, hd=hd, depth=depth,
                             n_mlp_chunks=n_mlp_chunks, eps=eps)
    fixed = lambda c, d: (0, 0)
    fixed3 = lambda c, d: (0, 0, 0)
    perblk = lambda c, d: (d, 0, 0)

    out = pl.pallas_call(
        kern,
        out_shape=jax.ShapeDtypeStruct((B, 1, D), jnp.float32),
        grid=(nc, depth),
        in_specs=[
            pl.BlockSpec((G, nP, K), lambda c, d: (c, 0, 0)),
            pl.BlockSpec((K, D), fixed),
            pl.BlockSpec((1, D), fixed),
            pl.BlockSpec((1, nP, D), fixed3),
            pl.BlockSpec((1, D), fixed),
            pl.BlockSpec((1, D), fixed),
            pl.BlockSpec((1, D), fixed),
            pl.BlockSpec((1, 1, D), perblk),
            pl.BlockSpec((1, 1, D), perblk),
            pl.BlockSpec((1, D, 3 * D), perblk),
            pl.BlockSpec((1, 1, 3 * D), perblk),
            pl.BlockSpec((1, D, D), perblk),
            pl.BlockSpec((1, 1, D), perblk),
            pl.BlockSpec((1, 1, D), perblk),
            pl.BlockSpec((1, 1, D), perblk),
            pl.BlockSpec((1, D, Hd), perblk),
            pl.BlockSpec((1, 1, Hd), perblk),
            pl.BlockSpec((1, Hd, D), perblk),
            pl.BlockSpec((1, 1, D), perblk),
        ],
        out_specs=pl.BlockSpec((G, 1, D), lambda c, d: (c, 0, 0)),
        scratch_shapes=[
            pltpu.VMEM((G, N, D), jnp.float32),
        ],
        compiler_params=pltpu.CompilerParams(
            dimension_semantics=("parallel", "arbitrary"),
            vmem_limit_bytes=63 * _MIB,
        ),
    )(xp, patch_w_t, patch_b, pos_patch, cls_full, norm_g, norm_b,
      stk["ln1_g"], stk["ln1_b"], stk["qkv_w"], stk["qkv_b"],
      stk["proj_w"], stk["proj_b"], stk["ln2_g"], stk["ln2_b"],
      stk["fc1_w"], stk["fc1_b"], stk["fc2_w"], stk["fc2_b"])
    return out[:, 0, :]


def kernel(x, patch_w_t, patch_b, cls_full, pos_patch, norm_g, norm_b, blk00_ln1_g, blk00_ln1_b, blk00_qkv_w_t, blk00_qkv_b, blk00_proj_w_t, blk00_proj_b, blk00_ln2_g, blk00_ln2_b, blk00_fc1_w_t, blk00_fc1_b, blk00_fc2_w_t, blk00_fc2_b, blk01_ln1_g, blk01_ln1_b, blk01_qkv_w_t, blk01_qkv_b, blk01_proj_w_t, blk01_proj_b, blk01_ln2_g, blk01_ln2_b, blk01_fc1_w_t, blk01_fc1_b, blk01_fc2_w_t, blk01_fc2_b, blk02_ln1_g, blk02_ln1_b, blk02_qkv_w_t, blk02_qkv_b, blk02_proj_w_t, blk02_proj_b, blk02_ln2_g, blk02_ln2_b, blk02_fc1_w_t, blk02_fc1_b, blk02_fc2_w_t, blk02_fc2_b, blk03_ln1_g, blk03_ln1_b, blk03_qkv_w_t, blk03_qkv_b, blk03_proj_w_t, blk03_proj_b, blk03_ln2_g, blk03_ln2_b, blk03_fc1_w_t, blk03_fc1_b, blk03_fc2_w_t, blk03_fc2_b, blk04_ln1_g, blk04_ln1_b, blk04_qkv_w_t, blk04_qkv_b, blk04_proj_w_t, blk04_proj_b, blk04_ln2_g, blk04_ln2_b, blk04_fc1_w_t, blk04_fc1_b, blk04_fc2_w_t, blk04_fc2_b, blk05_ln1_g, blk05_ln1_b, blk05_qkv_w_t, blk05_qkv_b, blk05_proj_w_t, blk05_proj_b, blk05_ln2_g, blk05_ln2_b, blk05_fc1_w_t, blk05_fc1_b, blk05_fc2_w_t, blk05_fc2_b, blk06_ln1_g, blk06_ln1_b, blk06_qkv_w_t, blk06_qkv_b, blk06_proj_w_t, blk06_proj_b, blk06_ln2_g, blk06_ln2_b, blk06_fc1_w_t, blk06_fc1_b, blk06_fc2_w_t, blk06_fc2_b, blk07_ln1_g, blk07_ln1_b, blk07_qkv_w_t, blk07_qkv_b, blk07_proj_w_t, blk07_proj_b, blk07_ln2_g, blk07_ln2_b, blk07_fc1_w_t, blk07_fc1_b, blk07_fc2_w_t, blk07_fc2_b, blk08_ln1_g, blk08_ln1_b, blk08_qkv_w_t, blk08_qkv_b, blk08_proj_w_t, blk08_proj_b, blk08_ln2_g, blk08_ln2_b, blk08_fc1_w_t, blk08_fc1_b, blk08_fc2_w_t, blk08_fc2_b, blk09_ln1_g, blk09_ln1_b, blk09_qkv_w_t, blk09_qkv_b, blk09_proj_w_t, blk09_proj_b, blk09_ln2_g, blk09_ln2_b, blk09_fc1_w_t, blk09_fc1_b, blk09_fc2_w_t, blk09_fc2_b, blk10_ln1_g, blk10_ln1_b, blk10_qkv_w_t, blk10_qkv_b, blk10_proj_w_t, blk10_proj_b, blk10_ln2_g, blk10_ln2_b, blk10_fc1_w_t, blk10_fc1_b, blk10_fc2_w_t, blk10_fc2_b, blk11_ln1_g, blk11_ln1_b, blk11_qkv_w_t, blk11_qkv_b, blk11_proj_w_t, blk11_proj_b, blk11_ln2_g, blk11_ln2_b, blk11_fc1_w_t, blk11_fc1_b, blk11_fc2_w_t, blk11_fc2_b):
    cfg_img, cfg_patch, cfg_cin = 224, 16, 3
    depth, nh = 12, 12
    B = x.shape[0]
    p = cfg_patch

    lcl = locals()
    names = ["ln1_g", "ln1_b", "qkv_w_t", "qkv_b", "proj_w_t", "proj_b",
             "ln2_g", "ln2_b", "fc1_w_t", "fc1_b", "fc2_w_t", "fc2_b"]
    blocks = [{n: lcl["blk%02d_%s" % (li, n)] for n in names}
              for li in range(depth)]

    D = patch_w_t.shape[1]
    scale = (D // nh) ** -0.5
    qsc_w = jnp.concatenate(
        [jnp.full((D,), scale, jnp.bfloat16),
         jnp.ones((2 * D,), jnp.bfloat16)])[None, None, :]
    qsc_b = jnp.concatenate(
        [jnp.full((D,), scale, jnp.float32),
         jnp.ones((2 * D,), jnp.float32)])[None, None, :]
    stk = {
        "ln1_g": jnp.stack([b["ln1_g"] for b in blocks]),
        "ln1_b": jnp.stack([b["ln1_b"] for b in blocks]),
        "qkv_w": jnp.stack([b["qkv_w_t"] for b in blocks]) * qsc_w,
        "qkv_b": jnp.stack([b["qkv_b"] for b in blocks]) * qsc_b,
        "proj_w": jnp.stack([b["proj_w_t"] for b in blocks]),
        "proj_b": jnp.stack([b["proj_b"] for b in blocks]),
        "ln2_g": jnp.stack([b["ln2_g"] for b in blocks]),
        "ln2_b": jnp.stack([b["ln2_b"] for b in blocks]),
        "fc1_w": jnp.stack([b["fc1_w_t"] for b in blocks]),
        "fc1_b": jnp.stack([b["fc1_b"] for b in blocks]),
        "fc2_w": jnp.stack([b["fc2_w_t"] for b in blocks]),
        "fc2_b": jnp.stack([b["fc2_b"] for b in blocks]),
    }

    H = W = cfg_img
    xp = x.reshape(B, cfg_cin, H // p, p, W // p, p)
    xp = xp.transpose(0, 2, 4, 1, 3, 5).reshape(
        B, (H // p) * (W // p), cfg_cin * p * p).astype(jnp.bfloat16)

    return _vit_forward(xp, patch_w_t, patch_b, pos_patch, cls_full,
                        norm_g, norm_b, stk, nh=nh, depth=depth, G=4)


# speedup vs baseline: 1.2527x; 1.1629x over previous
import functools
import math

import jax
import jax.numpy as jnp
from jax import lax
from jax.experimental import pallas as pl
from jax.experimental.pallas import tpu as pltpu

_MIB = 1024 * 1024


def _vit_megakernel(xp_ref, pw_ref, pb_ref, pos_ref, ng_ref, nb_ref,
                    ln1g_ref, ln1b_ref, qkvw_ref, qkvb_ref,
                    projw_ref, projb_ref, ln2g_ref, ln2b_ref,
                    fc1w_ref, fc1b_ref, fc2w_ref, fc2b_ref,
                    o_ref, scr,
                    *, G, nh, hd, depth, n_mlp_chunks, eps):
    d = pl.program_id(1)
    D = nh * hd

    @pl.when(d == 0)
    def _embed():
        def emb(i, _):
            y = jnp.dot(xp_ref[i], pw_ref[...],
                        preferred_element_type=jnp.float32)
            scr[i] = y + pb_ref[...] + pos_ref[0]
            return 0
        lax.fori_loop(0, G, emb, 0, unroll=True)

    ln1g = ln1g_ref[0]
    ln1b = ln1b_ref[0]
    wqkv = qkvw_ref[0]
    bqkv = qkvb_ref[0]
    wproj = projw_ref[0]
    bproj = projb_ref[0]
    ln2g = ln2g_ref[0]
    ln2b = ln2b_ref[0]
    w1 = fc1w_ref[0]
    b1 = fc1b_ref[0]
    w2 = fc2w_ref[0]
    b2 = fc2b_ref[0]
    Hc = w1.shape[1] // n_mlp_chunks

    inv_D = 1.0 / D

    def _ln(x, g, b):
        mu = jnp.sum(x, axis=-1, keepdims=True) * inv_D
        ms = jnp.sum(x * x, axis=-1, keepdims=True) * inv_D
        var = ms - mu * mu
        return (x - mu) * lax.rsqrt(var + eps) * g + b

    def body(i, _):
        x = scr[i]
        xb = _ln(x, ln1g, ln1b).astype(jnp.bfloat16)

        qkv = jnp.dot(xb, wqkv, preferred_element_type=jnp.float32) + bqkv
        qs = qkv[:, :D].astype(jnp.bfloat16)
        kb = qkv[:, D:2 * D].astype(jnp.bfloat16)
        vb = qkv[:, 2 * D:].astype(jnp.bfloat16)

        heads = []
        for h in range(nh):
            lo, hi = h * hd, (h + 1) * hd
            s = lax.dot_general(qs[:, lo:hi], kb[:, lo:hi],
                                (((1,), (1,)), ((), ())),
                                preferred_element_type=jnp.float32)
            p = jnp.exp(s)
            l = jnp.sum(p, axis=-1, keepdims=True)
            pv = jnp.dot(p.astype(jnp.bfloat16), vb[:, lo:hi],
                         preferred_element_type=jnp.float32)
            heads.append((pv * pl.reciprocal(l, approx=True)
                          ).astype(jnp.bfloat16))
        attn = jnp.concatenate(heads, axis=1)

        r1 = (x + jnp.dot(attn, wproj,
                          preferred_element_type=jnp.float32) + bproj)

        yn = _ln(r1, ln2g, ln2b).astype(jnp.bfloat16)

        hids = []
        for c in range(n_mlp_chunks):
            lo, hi = c * Hc, (c + 1) * Hc
            hid = (jnp.dot(yn, w1[:, lo:hi],
                           preferred_element_type=jnp.float32) + b1[:, lo:hi])
            hid = 0.5 * hid * (1.0 + lax.erf(hid * (1.0 / math.sqrt(2.0))))
            hids.append(hid.astype(jnp.bfloat16))
        hid_full = jnp.concatenate(hids, axis=1)
        scr[i] = r1 + b2 + jnp.dot(hid_full, w2,
                                   preferred_element_type=jnp.float32)
        return 0

    lax.fori_loop(0, G, body, 0, unroll=2)

    @pl.when(d == depth - 1)
    def _final():
        xc = scr[:, 0, :]
        o_ref[:, 0, :] = _ln(xc, ng_ref[...], nb_ref[...])


def _vit_forward(xp, patch_w_t, patch_b, pos_full, norm_g, norm_b,
                 stk, *, nh, depth, G, n_mlp_chunks=4, eps=1e-6):
    B, N, K = xp.shape
    D = patch_w_t.shape[1]
    Hd = stk["fc1_w"].shape[2]
    hd = D // nh
    nc = B // G

    kern = functools.partial(_vit_megakernel, G=G, nh=nh, hd=hd, depth=depth,
                             n_mlp_chunks=n_mlp_chunks, eps=eps)
    fixed = lambda c, d: (0, 0)
    fixed3 = lambda c, d: (0, 0, 0)
    perblk = lambda c, d: (d, 0, 0)

    out = pl.pallas_call(
        kern,
        out_shape=jax.ShapeDtypeStruct((B, 1, D), jnp.float32),
        grid=(nc, depth),
        in_specs=[
            pl.BlockSpec((G, N, K), lambda c, d: (c, 0, 0)),
            pl.BlockSpec((K, D), fixed),
            pl.BlockSpec((1, D), fixed),
            pl.BlockSpec((1, N, D), fixed3),
            pl.BlockSpec((1, D), fixed),
            pl.BlockSpec((1, D), fixed),
            pl.BlockSpec((1, 1, D), perblk),
            pl.BlockSpec((1, 1, D), perblk),
            pl.BlockSpec((1, D, 3 * D), perblk),
            pl.BlockSpec((1, 1, 3 * D), perblk),
            pl.BlockSpec((1, D, D), perblk),
            pl.BlockSpec((1, 1, D), perblk),
            pl.BlockSpec((1, 1, D), perblk),
            pl.BlockSpec((1, 1, D), perblk),
            pl.BlockSpec((1, D, Hd), perblk),
            pl.BlockSpec((1, 1, Hd), perblk),
            pl.BlockSpec((1, Hd, D), perblk),
            pl.BlockSpec((1, 1, D), perblk),
        ],
        out_specs=pl.BlockSpec((G, 1, D), lambda c, d: (c, 0, 0)),
        scratch_shapes=[
            pltpu.VMEM((G, N, D), jnp.float32),
        ],
        compiler_params=pltpu.CompilerParams(
            dimension_semantics=("parallel", "arbitrary"),
            vmem_limit_bytes=63 * _MIB,
        ),
    )(xp, patch_w_t, patch_b, pos_full, norm_g, norm_b,
      stk["ln1_g"], stk["ln1_b"], stk["qkv_w"], stk["qkv_b"],
      stk["proj_w"], stk["proj_b"], stk["ln2_g"], stk["ln2_b"],
      stk["fc1_w"], stk["fc1_b"], stk["fc2_w"], stk["fc2_b"])
    return out[:, 0, :]


def kernel(x, patch_w_t, patch_b, cls_full, pos_patch, norm_g, norm_b, blk00_ln1_g, blk00_ln1_b, blk00_qkv_w_t, blk00_qkv_b, blk00_proj_w_t, blk00_proj_b, blk00_ln2_g, blk00_ln2_b, blk00_fc1_w_t, blk00_fc1_b, blk00_fc2_w_t, blk00_fc2_b, blk01_ln1_g, blk01_ln1_b, blk01_qkv_w_t, blk01_qkv_b, blk01_proj_w_t, blk01_proj_b, blk01_ln2_g, blk01_ln2_b, blk01_fc1_w_t, blk01_fc1_b, blk01_fc2_w_t, blk01_fc2_b, blk02_ln1_g, blk02_ln1_b, blk02_qkv_w_t, blk02_qkv_b, blk02_proj_w_t, blk02_proj_b, blk02_ln2_g, blk02_ln2_b, blk02_fc1_w_t, blk02_fc1_b, blk02_fc2_w_t, blk02_fc2_b, blk03_ln1_g, blk03_ln1_b, blk03_qkv_w_t, blk03_qkv_b, blk03_proj_w_t, blk03_proj_b, blk03_ln2_g, blk03_ln2_b, blk03_fc1_w_t, blk03_fc1_b, blk03_fc2_w_t, blk03_fc2_b, blk04_ln1_g, blk04_ln1_b, blk04_qkv_w_t, blk04_qkv_b, blk04_proj_w_t, blk04_proj_b, blk04_ln2_g, blk04_ln2_b, blk04_fc1_w_t, blk04_fc1_b, blk04_fc2_w_t, blk04_fc2_b, blk05_ln1_g, blk05_ln1_b, blk05_qkv_w_t, blk05_qkv_b, blk05_proj_w_t, blk05_proj_b, blk05_ln2_g, blk05_ln2_b, blk05_fc1_w_t, blk05_fc1_b, blk05_fc2_w_t, blk05_fc2_b, blk06_ln1_g, blk06_ln1_b, blk06_qkv_w_t, blk06_qkv_b, blk06_proj_w_t, blk06_proj_b, blk06_ln2_g, blk06_ln2_b, blk06_fc1_w_t, blk06_fc1_b, blk06_fc2_w_t, blk06_fc2_b, blk07_ln1_g, blk07_ln1_b, blk07_qkv_w_t, blk07_qkv_b, blk07_proj_w_t, blk07_proj_b, blk07_ln2_g, blk07_ln2_b, blk07_fc1_w_t, blk07_fc1_b, blk07_fc2_w_t, blk07_fc2_b, blk08_ln1_g, blk08_ln1_b, blk08_qkv_w_t, blk08_qkv_b, blk08_proj_w_t, blk08_proj_b, blk08_ln2_g, blk08_ln2_b, blk08_fc1_w_t, blk08_fc1_b, blk08_fc2_w_t, blk08_fc2_b, blk09_ln1_g, blk09_ln1_b, blk09_qkv_w_t, blk09_qkv_b, blk09_proj_w_t, blk09_proj_b, blk09_ln2_g, blk09_ln2_b, blk09_fc1_w_t, blk09_fc1_b, blk09_fc2_w_t, blk09_fc2_b, blk10_ln1_g, blk10_ln1_b, blk10_qkv_w_t, blk10_qkv_b, blk10_proj_w_t, blk10_proj_b, blk10_ln2_g, blk10_ln2_b, blk10_fc1_w_t, blk10_fc1_b, blk10_fc2_w_t, blk10_fc2_b, blk11_ln1_g, blk11_ln1_b, blk11_qkv_w_t, blk11_qkv_b, blk11_proj_w_t, blk11_proj_b, blk11_ln2_g, blk11_ln2_b, blk11_fc1_w_t, blk11_fc1_b, blk11_fc2_w_t, blk11_fc2_b):
    cfg_img, cfg_patch, cfg_cin = 224, 16, 3
    depth, nh = 12, 12
    B = x.shape[0]
    p = cfg_patch

    lcl = locals()
    names = ["ln1_g", "ln1_b", "qkv_w_t", "qkv_b", "proj_w_t", "proj_b",
             "ln2_g", "ln2_b", "fc1_w_t", "fc1_b", "fc2_w_t", "fc2_b"]
    blocks = [{n: lcl["blk%02d_%s" % (li, n)] for n in names}
              for li in range(depth)]

    D = patch_w_t.shape[1]
    scale = (D // nh) ** -0.5
    qsc_w = jnp.concatenate(
        [jnp.full((D,), scale, jnp.bfloat16),
         jnp.ones((2 * D,), jnp.bfloat16)])[None, None, :]
    qsc_b = jnp.concatenate(
        [jnp.full((D,), scale, jnp.float32),
         jnp.ones((2 * D,), jnp.float32)])[None, None, :]
    stk = {
        "ln1_g": jnp.stack([b["ln1_g"] for b in blocks]),
        "ln1_b": jnp.stack([b["ln1_b"] for b in blocks]),
        "qkv_w": jnp.stack([b["qkv_w_t"] for b in blocks]) * qsc_w,
        "qkv_b": jnp.stack([b["qkv_b"] for b in blocks]) * qsc_b,
        "proj_w": jnp.stack([b["proj_w_t"] for b in blocks]),
        "proj_b": jnp.stack([b["proj_b"] for b in blocks]),
        "ln2_g": jnp.stack([b["ln2_g"] for b in blocks]),
        "ln2_b": jnp.stack([b["ln2_b"] for b in blocks]),
        "fc1_w": jnp.stack([b["fc1_w_t"] for b in blocks]),
        "fc1_b": jnp.stack([b["fc1_b"] for b in blocks]),
        "fc2_w": jnp.stack([b["fc2_w_t"] for b in blocks]),
        "fc2_b": jnp.stack([b["fc2_b"] for b in blocks]),
    }

    H = W = cfg_img
    xp = x.reshape(B, cfg_cin, H // p, p, W // p, p)
    xp = xp.transpose(0, 2, 4, 1, 3, 5).reshape(
        B, (H // p) * (W // p), cfg_cin * p * p).astype(jnp.bfloat16)
    xp = jnp.pad(xp, ((0, 0), (1, 0), (0, 0)))
    pos_full = jnp.concatenate([(cls_full - patch_b)[None], pos_patch], axis=1)

    return _vit_forward(xp, patch_w_t, patch_b, pos_full,
                        norm_g, norm_b, stk, nh=nh, depth=depth, G=4)
